# baseline (device time: 294901 ns/iter reference)
import jax
import jax.numpy as jnp
from jax import lax
from jax.experimental import pallas as pl
from jax.experimental.pallas import tpu as pltpu

N_DEV = 8
B_PER = 2
SQ = 512
D_MODEL = 768
HQ = 64
DH = 64
HEADS_PER = 8
BLK = 64


def _ring_allgather_weights(wq, wo):

    def body(wq_ref, wo_ref, wq_all, wo_all, ss_q, rs_q, ss_o, rs_o):
        my = lax.axis_index("i")
        left = lax.rem(my + N_DEV - 1, N_DEV)
        right = lax.rem(my + 1, N_DEV)

        barrier_sem = pltpu.get_barrier_semaphore()
        for nbr in (left, right):
            pl.semaphore_signal(
                barrier_sem, inc=1,
                device_id=(nbr,), device_id_type=pl.DeviceIdType.MESH,
            )
        pl.semaphore_wait(barrier_sem, 2)

        wq_all[pl.ds(my, 1)] = wq_ref[:, :][None]
        wo_all[pl.ds(my, 1)] = wo_ref[:, :][None]

        for h in range(N_DEV - 1):
            src_o = lax.rem(my - h + 2 * N_DEV, N_DEV)
            rcv_o = lax.rem(my - 1 - h + 2 * N_DEV, N_DEV)

            sends = []
            recvs = []
            for all_ref, ss, rs in ((wq_all, ss_q, rs_q), (wo_all, ss_o, rs_o)):
                send = pltpu.make_async_remote_copy(
                    src_ref=all_ref.at[src_o],
                    dst_ref=all_ref.at[src_o],
                    send_sem=ss.at[h],
                    recv_sem=rs.at[h],
                    device_id=(right,),
                    device_id_type=pl.DeviceIdType.MESH,
                )
                send.start()
                sends.append(send)
                recv = pltpu.make_async_remote_copy(
                    src_ref=all_ref.at[rcv_o],
                    dst_ref=all_ref.at[rcv_o],
                    send_sem=ss.at[h],
                    recv_sem=rs.at[h],
                    device_id=(left,),
                    device_id_type=pl.DeviceIdType.MESH,
                )
                recvs.append(recv)
            for r in recvs:
                r.wait_recv()
            for s in sends:
                s.wait_send()

    return pl.pallas_call(
        body,
        out_shape=[
            jax.ShapeDtypeStruct((N_DEV, D_MODEL, HEADS_PER * DH), jnp.bfloat16),
            jax.ShapeDtypeStruct((N_DEV, HEADS_PER * DH, D_MODEL), jnp.bfloat16),
        ],
        in_specs=[
            pl.BlockSpec(memory_space=pltpu.VMEM),
            pl.BlockSpec(memory_space=pltpu.VMEM),
        ],
        out_specs=[
            pl.BlockSpec(memory_space=pltpu.VMEM),
            pl.BlockSpec(memory_space=pltpu.VMEM),
        ],
        scratch_shapes=[
            pltpu.SemaphoreType.DMA((N_DEV - 1,)),
            pltpu.SemaphoreType.DMA((N_DEV - 1,)),
            pltpu.SemaphoreType.DMA((N_DEV - 1,)),
            pltpu.SemaphoreType.DMA((N_DEV - 1,)),
        ],
        compiler_params=pltpu.CompilerParams(collective_id=0),
    )(wq, wo)


def kernel(x, Wq, K_ext, V_ext, Wo):
    wq_all, wo_all = _ring_allgather_weights(
        Wq.astype(jnp.bfloat16), Wo.astype(jnp.bfloat16)
    )
    wq_full = wq_all.transpose(1, 0, 2).reshape(D_MODEL, HQ * DH)
    wo_full = wo_all.reshape(HQ * DH, D_MODEL)

    my = lax.axis_index("i")
    kb = lax.dynamic_slice_in_dim(K_ext, my * B_PER, B_PER, 0).astype(jnp.bfloat16)
    vb = lax.dynamic_slice_in_dim(V_ext, my * B_PER, B_PER, 0).astype(jnp.bfloat16)
    xb = x.astype(jnp.bfloat16)

    q = (xb.reshape(B_PER * SQ, D_MODEL) @ wq_full).reshape(B_PER, SQ, HQ, DH)
    scores = (
        jnp.einsum("bihd,bjhd->bhij", q, kb, preferred_element_type=jnp.float32)
        * 0.125
    )
    blk_q = (jnp.arange(SQ) // BLK)[:, None]
    blk_k = (jnp.arange(SQ) // BLK)[None, :]
    mask = blk_k <= blk_q
    scores = jnp.where(mask[None, None], scores, -1e9)
    m = scores.max(axis=-1, keepdims=True)
    w = jnp.exp(scores - m)
    w = w / w.sum(axis=-1, keepdims=True)
    ctx = jnp.einsum(
        "bhij,bjhd->bihd", w.astype(jnp.bfloat16), vb
    ).reshape(B_PER * SQ, HQ * DH)
    out = jnp.dot(ctx, wo_full, preferred_element_type=jnp.float32)
    return out.reshape(B_PER, SQ, D_MODEL)


# device time: 247442 ns/iter; 1.1918x vs baseline; 1.1918x over previous
import jax
import jax.numpy as jnp
from jax import lax
from jax.experimental import pallas as pl
from jax.experimental.pallas import tpu as pltpu

N_DEV = 8
B_PER = 2
SQ = 512
D_MODEL = 768
HQ = 64
DH = 64
HEADS_PER = 8
BLK = 64
NEG = -1e9


def _fused(xb, wq, wo, kt, vt):

    def compute_block(o, mask, x_ref, kt_ref, vt_ref,
                      wq_all, wo_all, q_ref, ctx_ref, out_ref):
        wq_o = wq_all[pl.ds(o, 1)].reshape(D_MODEL, HEADS_PER * DH)
        wo_o = wo_all[pl.ds(o, 1)].reshape(HEADS_PER * DH, D_MODEL)
        q_ref[...] = jnp.dot(
            x_ref[...], wq_o, preferred_element_type=jnp.float32
        ).astype(jnp.bfloat16)
        for b in range(B_PER):
            for hh in range(HEADS_PER):
                bh = b * HQ + o * HEADS_PER + hh
                k_h = kt_ref[pl.ds(bh * DH, DH), :]
                v_h = vt_ref[pl.ds(bh * DH, DH), :]
                q_h = q_ref[b * SQ:(b + 1) * SQ, hh * DH:(hh + 1) * DH]
                s = jnp.dot(q_h, k_h, preferred_element_type=jnp.float32)
                s = jnp.where(mask, s * 0.125, NEG)
                m = jnp.max(s, axis=-1, keepdims=True)
                w = jnp.exp(s - m)
                w = (w / jnp.sum(w, axis=-1, keepdims=True)).astype(jnp.bfloat16)
                c = lax.dot_general(
                    w, v_h, (((1,), (1,)), ((), ())),
                    preferred_element_type=jnp.float32,
                )
                ctx_ref[b * SQ:(b + 1) * SQ, hh * DH:(hh + 1) * DH] = (
                    c.astype(jnp.bfloat16)
                )
        out_ref[...] += jnp.dot(
            ctx_ref[...], wo_o, preferred_element_type=jnp.float32
        ).reshape(B_PER, SQ, D_MODEL)

    def body(x_ref, wq_ref, wo_ref, kt_ref, vt_ref, out_ref,
             wq_all, wo_all, q_ref, ctx_ref, ss_q, rs_q, ss_o, rs_o):
        my = lax.axis_index("i")
        left = lax.rem(my + N_DEV - 1, N_DEV)
        right = lax.rem(my + 1, N_DEV)

        barrier_sem = pltpu.get_barrier_semaphore()
        for nbr in (left, right):
            pl.semaphore_signal(
                barrier_sem, inc=1,
                device_id=(nbr,), device_id_type=pl.DeviceIdType.MESH,
            )
        pl.semaphore_wait(barrier_sem, 2)

        wq_all[pl.ds(my, 1)] = wq_ref[:, :][None]
        wo_all[pl.ds(my, 1)] = wo_ref[:, :][None]
        out_ref[...] = jnp.zeros((B_PER, SQ, D_MODEL), jnp.float32)

        blk_i = lax.broadcasted_iota(jnp.int32, (SQ, SQ), 0) // BLK
        blk_j = lax.broadcasted_iota(jnp.int32, (SQ, SQ), 1) // BLK
        mask = blk_j <= blk_i

        args = (mask, x_ref, kt_ref, vt_ref, wq_all, wo_all, q_ref, ctx_ref,
                out_ref)

        compute_block(my, *args)

        for h in range(N_DEV - 1):
            src_o = lax.rem(my - h + 2 * N_DEV, N_DEV)
            rcv_o = lax.rem(my - 1 - h + 2 * N_DEV, N_DEV)
            sends, recvs = [], []
            for all_ref, ss, rs in ((wq_all, ss_q, rs_q), (wo_all, ss_o, rs_o)):
                send = pltpu.make_async_remote_copy(
                    src_ref=all_ref.at[src_o], dst_ref=all_ref.at[src_o],
                    send_sem=ss.at[h], recv_sem=rs.at[h],
                    device_id=(right,), device_id_type=pl.DeviceIdType.MESH,
                )
                send.start()
                sends.append(send)
                recv = pltpu.make_async_remote_copy(
                    src_ref=all_ref.at[rcv_o], dst_ref=all_ref.at[rcv_o],
                    send_sem=ss.at[h], recv_sem=rs.at[h],
                    device_id=(left,), device_id_type=pl.DeviceIdType.MESH,
                )
                recvs.append(recv)
            for r in recvs:
                r.wait_recv()
            for s in sends:
                s.wait_send()
            compute_block(rcv_o, *args)

    return pl.pallas_call(
        body,
        out_shape=jax.ShapeDtypeStruct((B_PER, SQ, D_MODEL), jnp.float32),
        in_specs=[pl.BlockSpec(memory_space=pltpu.VMEM)] * 5,
        out_specs=pl.BlockSpec(memory_space=pltpu.VMEM),
        scratch_shapes=[
            pltpu.VMEM((N_DEV, D_MODEL, HEADS_PER * DH), jnp.bfloat16),
            pltpu.VMEM((N_DEV, HEADS_PER * DH, D_MODEL), jnp.bfloat16),
            pltpu.VMEM((B_PER * SQ, HEADS_PER * DH), jnp.bfloat16),
            pltpu.VMEM((B_PER * SQ, HEADS_PER * DH), jnp.bfloat16),
            pltpu.SemaphoreType.DMA((N_DEV - 1,)),
            pltpu.SemaphoreType.DMA((N_DEV - 1,)),
            pltpu.SemaphoreType.DMA((N_DEV - 1,)),
            pltpu.SemaphoreType.DMA((N_DEV - 1,)),
        ],
        compiler_params=pltpu.CompilerParams(
            collective_id=0, vmem_limit_bytes=63 * 1024 * 1024
        ),
    )(xb, wq, wo, kt, vt)


def kernel(x, Wq, K_ext, V_ext, Wo):
    my = lax.axis_index("i")
    kb = lax.dynamic_slice_in_dim(K_ext, my * B_PER, B_PER, 0)
    vb = lax.dynamic_slice_in_dim(V_ext, my * B_PER, B_PER, 0)
    kt = jnp.transpose(kb, (0, 2, 3, 1)).astype(jnp.bfloat16).reshape(
        B_PER * HQ * DH, SQ
    )
    vt = jnp.transpose(vb, (0, 2, 3, 1)).astype(jnp.bfloat16).reshape(
        B_PER * HQ * DH, SQ
    )
    xb = x.astype(jnp.bfloat16).reshape(B_PER * SQ, D_MODEL)
    return _fused(
        xb, Wq.astype(jnp.bfloat16), Wo.astype(jnp.bfloat16), kt, vt
    )


# device time: 181726 ns/iter; 1.6228x vs baseline; 1.3616x over previous
import jax
import jax.numpy as jnp
from jax import lax
from jax.experimental import pallas as pl
from jax.experimental.pallas import tpu as pltpu

N_DEV = 8
B_PER = 2
SQ = 512
D_MODEL = 768
HQ = 64
DH = 64
HEADS_PER = 8
BLK = 64
NEG = -1e9


def _fused(xb, wq, wo, kt, vt):

    def compute_block(o, bias, x_ref, kt_ref, vt_ref,
                      wq_all, wo_all, q_ref, ctx_ref, out_ref):
        wq_o = wq_all[pl.ds(o, 1)].reshape(D_MODEL, HEADS_PER * DH)
        wo_o = wo_all[pl.ds(o, 1)].reshape(HEADS_PER * DH, D_MODEL)
        q_ref[...] = jnp.dot(
            x_ref[...], wq_o, preferred_element_type=jnp.float32
        ).astype(jnp.bfloat16)
        for b in range(B_PER):
            for hh in range(HEADS_PER):
                bh = b * HQ + o * HEADS_PER + hh
                k_h = kt_ref[pl.ds(bh * DH, DH), :]
                v_h = vt_ref[pl.ds(bh * DH, DH), :]
                q_h = q_ref[b * SQ:(b + 1) * SQ, hh * DH:(hh + 1) * DH]
                s = jnp.dot(q_h, k_h, preferred_element_type=jnp.float32)
                w = jnp.exp(s * 0.125 + bias)
                r = 1.0 / jnp.sum(w, axis=-1, keepdims=True)
                c = lax.dot_general(
                    w.astype(jnp.bfloat16), v_h, (((1,), (1,)), ((), ())),
                    preferred_element_type=jnp.float32,
                )
                ctx_ref[b * SQ:(b + 1) * SQ, hh * DH:(hh + 1) * DH] = (
                    (c * r).astype(jnp.bfloat16)
                )
        out_ref[...] += jnp.dot(
            ctx_ref[...], wo_o, preferred_element_type=jnp.float32
        ).reshape(B_PER, SQ, D_MODEL)

    def body(x_ref, wq_ref, wo_ref, kt_ref, vt_ref, out_ref,
             wq_all, wo_all, q_ref, ctx_ref, ss_q, rs_q, ss_o, rs_o):
        my = lax.axis_index("i")
        left = lax.rem(my + N_DEV - 1, N_DEV)
        right = lax.rem(my + 1, N_DEV)

        barrier_sem = pltpu.get_barrier_semaphore()
        for nbr in (left, right):
            pl.semaphore_signal(
                barrier_sem, inc=1,
                device_id=(nbr,), device_id_type=pl.DeviceIdType.MESH,
            )
        pl.semaphore_wait(barrier_sem, 2)

        wq_all[pl.ds(my, 1)] = wq_ref[:, :][None]
        wo_all[pl.ds(my, 1)] = wo_ref[:, :][None]
        out_ref[...] = jnp.zeros((B_PER, SQ, D_MODEL), jnp.float32)

        blk_i = lax.broadcasted_iota(jnp.int32, (SQ, SQ), 0) // BLK
        blk_j = lax.broadcasted_iota(jnp.int32, (SQ, SQ), 1) // BLK
        bias = jnp.where(blk_j <= blk_i, 0.0, NEG).astype(jnp.float32)

        args = (bias, x_ref, kt_ref, vt_ref, wq_all, wo_all, q_ref, ctx_ref,
                out_ref)

        def start_hop(h):
            src_o = lax.rem(my - h + 2 * N_DEV, N_DEV)
            rcv_o = lax.rem(my - 1 - h + 2 * N_DEV, N_DEV)
            sends, recvs = [], []
            for all_ref, ss, rs in ((wq_all, ss_q, rs_q), (wo_all, ss_o, rs_o)):
                send = pltpu.make_async_remote_copy(
                    src_ref=all_ref.at[src_o], dst_ref=all_ref.at[src_o],
                    send_sem=ss.at[h], recv_sem=rs.at[h],
                    device_id=(right,), device_id_type=pl.DeviceIdType.MESH,
                )
                send.start()
                sends.append(send)
                recv = pltpu.make_async_remote_copy(
                    src_ref=all_ref.at[rcv_o], dst_ref=all_ref.at[rcv_o],
                    send_sem=ss.at[h], recv_sem=rs.at[h],
                    device_id=(left,), device_id_type=pl.DeviceIdType.MESH,
                )
                recvs.append(recv)
            return sends, recvs

        pending_sends = []
        sends, recvs = start_hop(0)
        pending_sends += sends
        compute_block(my, *args)
        for h in range(N_DEV - 1):
            for r in recvs:
                r.wait_recv()
            rcv_o = lax.rem(my - 1 - h + 2 * N_DEV, N_DEV)
            if h + 1 < N_DEV - 1:
                sends, recvs = start_hop(h + 1)
                pending_sends += sends
            compute_block(rcv_o, *args)
        for s in pending_sends:
            s.wait_send()

    return pl.pallas_call(
        body,
        out_shape=jax.ShapeDtypeStruct((B_PER, SQ, D_MODEL), jnp.float32),
        in_specs=[pl.BlockSpec(memory_space=pltpu.VMEM)] * 5,
        out_specs=pl.BlockSpec(memory_space=pltpu.VMEM),
        scratch_shapes=[
            pltpu.VMEM((N_DEV, D_MODEL, HEADS_PER * DH), jnp.bfloat16),
            pltpu.VMEM((N_DEV, HEADS_PER * DH, D_MODEL), jnp.bfloat16),
            pltpu.VMEM((B_PER * SQ, HEADS_PER * DH), jnp.bfloat16),
            pltpu.VMEM((B_PER * SQ, HEADS_PER * DH), jnp.bfloat16),
            pltpu.SemaphoreType.DMA((N_DEV - 1,)),
            pltpu.SemaphoreType.DMA((N_DEV - 1,)),
            pltpu.SemaphoreType.DMA((N_DEV - 1,)),
            pltpu.SemaphoreType.DMA((N_DEV - 1,)),
        ],
        compiler_params=pltpu.CompilerParams(
            collective_id=0, vmem_limit_bytes=63 * 1024 * 1024
        ),
    )(xb, wq, wo, kt, vt)


def kernel(x, Wq, K_ext, V_ext, Wo):
    my = lax.axis_index("i")
    kb = lax.dynamic_slice_in_dim(K_ext, my * B_PER, B_PER, 0)
    vb = lax.dynamic_slice_in_dim(V_ext, my * B_PER, B_PER, 0)
    kt = jnp.transpose(kb, (0, 2, 3, 1)).astype(jnp.bfloat16).reshape(
        B_PER * HQ * DH, SQ
    )
    vt = jnp.transpose(vb, (0, 2, 3, 1)).astype(jnp.bfloat16).reshape(
        B_PER * HQ * DH, SQ
    )
    xb = x.astype(jnp.bfloat16).reshape(B_PER * SQ, D_MODEL)
    return _fused(
        xb, Wq.astype(jnp.bfloat16), Wo.astype(jnp.bfloat16), kt, vt
    )


# device time: 125942 ns/iter; 2.3416x vs baseline; 1.4429x over previous
import jax
import jax.numpy as jnp
from jax import lax
from jax.experimental import pallas as pl
from jax.experimental.pallas import tpu as pltpu

N_DEV = 8
B_PER = 2
SQ = 512
D_MODEL = 768
HQ = 64
DH = 64
HEADS_PER = 8
BLK = 64
NEG = -1e9


def _fused(xb, wq, wo, kt, vt):

    def compute_block(o, bias, x_ref, kt_ref, vt_ref,
                      wq_all, wo_all, q_ref, ctx_ref, out_ref):
        wq_o = wq_all[pl.ds(o, 1)].reshape(D_MODEL, HEADS_PER * DH)
        wo_o = wo_all[pl.ds(o, 1)].reshape(HEADS_PER * DH, D_MODEL)
        q_ref[...] = jnp.dot(
            x_ref[...], wq_o, preferred_element_type=jnp.float32
        ).astype(jnp.bfloat16)
        for b in range(B_PER):
            for hh in range(HEADS_PER):
                bh = b * HQ + o * HEADS_PER + hh
                k_h = kt_ref[pl.ds(bh * DH, DH), :]
                v_h = vt_ref[pl.ds(bh * DH, DH), :]
                q_h = q_ref[b * SQ:(b + 1) * SQ, hh * DH:(hh + 1) * DH]
                s = jnp.dot(q_h, k_h, preferred_element_type=jnp.float32)
                w = jnp.exp(s * 0.125 + bias)
                r = 1.0 / jnp.sum(w, axis=-1, keepdims=True)
                c = lax.dot_general(
                    w.astype(jnp.bfloat16), v_h, (((1,), (1,)), ((), ())),
                    preferred_element_type=jnp.float32,
                )
                ctx_ref[b * SQ:(b + 1) * SQ, hh * DH:(hh + 1) * DH] = (
                    (c * r).astype(jnp.bfloat16)
                )
        out_ref[...] += jnp.dot(
            ctx_ref[...], wo_o, preferred_element_type=jnp.float32
        ).reshape(B_PER, SQ, D_MODEL)

    def body(x_ref, wq_ref, wo_ref, kt_ref, vt_ref, out_ref,
             wq_all, wo_all, q_ref, ctx_ref, ss_r, rs_r, ss_l, rs_l):
        my = lax.axis_index("i")
        left = lax.rem(my + N_DEV - 1, N_DEV)
        right = lax.rem(my + 1, N_DEV)

        barrier_sem = pltpu.get_barrier_semaphore()
        for nbr in (left, right):
            pl.semaphore_signal(
                barrier_sem, inc=1,
                device_id=(nbr,), device_id_type=pl.DeviceIdType.MESH,
            )
        pl.semaphore_wait(barrier_sem, 2)

        wq_all[pl.ds(my, 1)] = wq_ref[:, :][None]
        wo_all[pl.ds(my, 1)] = wo_ref[:, :][None]
        out_ref[...] = jnp.zeros((B_PER, SQ, D_MODEL), jnp.float32)

        blk_i = lax.broadcasted_iota(jnp.int32, (SQ, SQ), 0) // BLK
        blk_j = lax.broadcasted_iota(jnp.int32, (SQ, SQ), 1) // BLK
        bias = jnp.where(blk_j <= blk_i, 0.0, NEG).astype(jnp.float32)

        args = (bias, x_ref, kt_ref, vt_ref, wq_all, wo_all, q_ref, ctx_ref,
                out_ref)

        R_HOPS = N_DEV // 2
        L_HOPS = N_DEV - 1 - R_HOPS

        def start_hop(r, to_right):
            if to_right:
                src_o = lax.rem(my - r + 2 * N_DEV, N_DEV)
                rcv_o = lax.rem(my - 1 - r + 2 * N_DEV, N_DEV)
                dst, ss2, rs2 = right, ss_r, rs_r
            else:
                src_o = lax.rem(my + r, N_DEV)
                rcv_o = lax.rem(my + 1 + r, N_DEV)
                dst, ss2, rs2 = left, ss_l, rs_l
            sends, recvs = [], []
            for t, all_ref in enumerate((wq_all, wo_all)):
                send = pltpu.make_async_remote_copy(
                    src_ref=all_ref.at[src_o], dst_ref=all_ref.at[src_o],
                    send_sem=ss2.at[t, r], recv_sem=rs2.at[t, r],
                    device_id=(dst,), device_id_type=pl.DeviceIdType.MESH,
                )
                send.start()
                sends.append(send)
                recv = pltpu.make_async_remote_copy(
                    src_ref=all_ref.at[rcv_o], dst_ref=all_ref.at[rcv_o],
                    send_sem=ss2.at[t, r], recv_sem=rs2.at[t, r],
                    device_id=(dst,), device_id_type=pl.DeviceIdType.MESH,
                )
                recvs.append(recv)
            return sends, recvs

        pending_sends = []
        sr, recvs_r = start_hop(0, True)
        sl, recvs_l = start_hop(0, False)
        pending_sends += sr + sl
        compute_block(my, *args)
        for r in range(R_HOPS):
            for d in recvs_r:
                d.wait_recv()
            rcv_right = lax.rem(my - 1 - r + 2 * N_DEV, N_DEV)
            if r + 1 < R_HOPS:
                sr, recvs_r = start_hop(r + 1, True)
                pending_sends += sr
            if r < L_HOPS:
                for d in recvs_l:
                    d.wait_recv()
                rcv_left = lax.rem(my + 1 + r, N_DEV)
                if r + 1 < L_HOPS:
                    sl, recvs_l = start_hop(r + 1, False)
                    pending_sends += sl
                compute_block(rcv_left, *args)
            compute_block(rcv_right, *args)
        for s in pending_sends:
            s.wait_send()

    return pl.pallas_call(
        body,
        out_shape=jax.ShapeDtypeStruct((B_PER, SQ, D_MODEL), jnp.float32),
        in_specs=[pl.BlockSpec(memory_space=pltpu.VMEM)] * 5,
        out_specs=pl.BlockSpec(memory_space=pltpu.VMEM),
        scratch_shapes=[
            pltpu.VMEM((N_DEV, D_MODEL, HEADS_PER * DH), jnp.bfloat16),
            pltpu.VMEM((N_DEV, HEADS_PER * DH, D_MODEL), jnp.bfloat16),
            pltpu.VMEM((B_PER * SQ, HEADS_PER * DH), jnp.bfloat16),
            pltpu.VMEM((B_PER * SQ, HEADS_PER * DH), jnp.bfloat16),
            pltpu.SemaphoreType.DMA((2, N_DEV // 2)),
            pltpu.SemaphoreType.DMA((2, N_DEV // 2)),
            pltpu.SemaphoreType.DMA((2, N_DEV - 1 - N_DEV // 2)),
            pltpu.SemaphoreType.DMA((2, N_DEV - 1 - N_DEV // 2)),
        ],
        compiler_params=pltpu.CompilerParams(
            collective_id=0, vmem_limit_bytes=63 * 1024 * 1024
        ),
    )(xb, wq, wo, kt, vt)


def kernel(x, Wq, K_ext, V_ext, Wo):
    my = lax.axis_index("i")
    kb = lax.dynamic_slice_in_dim(K_ext, my * B_PER, B_PER, 0)
    vb = lax.dynamic_slice_in_dim(V_ext, my * B_PER, B_PER, 0)
    kt = jnp.transpose(kb, (0, 2, 3, 1)).astype(jnp.bfloat16).reshape(
        B_PER * HQ * DH, SQ
    )
    vt = jnp.transpose(vb, (0, 2, 3, 1)).astype(jnp.bfloat16).reshape(
        B_PER * HQ * DH, SQ
    )
    xb = x.astype(jnp.bfloat16).reshape(B_PER * SQ, D_MODEL)
    return _fused(
        xb, Wq.astype(jnp.bfloat16), Wo.astype(jnp.bfloat16), kt, vt
    )
